# baseline (speedup 1.0000x reference)
import functools

import jax
import jax.numpy as jnp
from jax import lax
from jax.experimental import pallas as pl
from jax.experimental.pallas import tpu as pltpu
from jax.experimental.pallas import tpu_sc as plsc

NUM_NODES = 100000
E = 320000
D = 128
D_EDGE = 16
N_TYPES = 8

NTILES = 32
NPAD = 100096
EPW = E // NTILES
RPW = NPAD // NTILES


def _mesh():
    return plsc.VectorSubcoreMesh(core_axis_name="c", subcore_axis_name="s")


def _wid():
    return lax.axis_index("s") * 2 + lax.axis_index("c")


def _ring_gather_job(tab, idxref, out, base, C, nchunk, buf_a, buf_b,
                     sem_a, sem_b):
    def sg(j, buf, sem):
        pltpu.async_copy(tab.at[idxref.at[pl.ds(j * C, C)]], buf, sem)

    def wg(buf, sem):
        pltpu.make_async_copy(tab.at[pl.ds(0, C)], buf, sem).wait()

    def out_cp(j, buf):
        pltpu.sync_copy(buf, out.at[pl.ds(base + j * C, C)])

    sg(0, buf_a, sem_a)
    sg(1, buf_b, sem_b)
    npair = (nchunk - 2) // 2 if nchunk % 2 == 0 else (nchunk - 3) // 2

    def body(j2, _):
        j = 2 * j2
        wg(buf_a, sem_a)
        out_cp(j, buf_a)
        sg(j + 2, buf_a, sem_a)
        wg(buf_b, sem_b)
        out_cp(j + 1, buf_b)
        sg(j + 3, buf_b, sem_b)
        return 0

    lax.fori_loop(0, npair, body, 0)
    if nchunk % 2 == 0:
        wg(buf_a, sem_a)
        out_cp(nchunk - 2, buf_a)
        wg(buf_b, sem_b)
        out_cp(nchunk - 1, buf_b)
    else:
        wg(buf_a, sem_a)
        out_cp(nchunk - 3, buf_a)
        sg(nchunk - 1, buf_a, sem_a)
        wg(buf_b, sem_b)
        out_cp(nchunk - 2, buf_b)
        wg(buf_a, sem_a)
        out_cp(nchunk - 1, buf_a)


def _ring_gather_multi(jobs, base, C, nchunk):
    def sg(tab, idxref, j, buf, sem):
        pltpu.async_copy(tab.at[idxref.at[pl.ds(j * C, C)]], buf, sem)

    def wg(tab, buf, sem):
        pltpu.make_async_copy(tab.at[pl.ds(0, C)], buf, sem).wait()

    def out_cp(out, j, buf):
        pltpu.sync_copy(buf, out.at[pl.ds(base + j * C, C)])

    for tab, idx, out, ba, bb, sa, sb in jobs:
        sg(tab, idx, 0, ba, sa)
        sg(tab, idx, 1, bb, sb)
    npair = (nchunk - 2) // 2 if nchunk % 2 == 0 else (nchunk - 3) // 2

    def body(j2, _):
        j = 2 * j2
        for tab, idx, out, ba, bb, sa, sb in jobs:
            wg(tab, ba, sa)
            out_cp(out, j, ba)
            sg(tab, idx, j + 2, ba, sa)
            wg(tab, bb, sb)
            out_cp(out, j + 1, bb)
            sg(tab, idx, j + 3, bb, sb)
        return 0

    lax.fori_loop(0, npair, body, 0)
    for tab, idx, out, ba, bb, sa, sb in jobs:
        if nchunk % 2 == 0:
            wg(tab, ba, sa)
            out_cp(out, nchunk - 2, ba)
            wg(tab, bb, sb)
            out_cp(out, nchunk - 1, bb)
        else:
            wg(tab, ba, sa)
            out_cp(out, nchunk - 3, ba)
            sg(tab, idx, nchunk - 1, ba, sa)
            wg(tab, bb, sb)
            out_cp(out, nchunk - 2, bb)
            wg(tab, ba, sa)
            out_cp(out, nchunk - 1, ba)


def _sc_winner_tables(src, dst):
    grp = EPW // 16

    @functools.partial(
        pl.kernel,
        mesh=_mesh(),
        compiler_params=pltpu.CompilerParams(needs_layout_passes=False),
        out_type=[
            jax.ShapeDtypeStruct((NTILES, NPAD), jnp.int32),
            jax.ShapeDtypeStruct((NTILES, NPAD), jnp.int32),
        ],
        scratch_types=[
            pltpu.VMEM((NPAD,), jnp.int32),
            pltpu.VMEM((EPW,), jnp.int32),
        ],
    )
    def k(src_hbm, dst_hbm, ls_hbm, ld_hbm, tbl, chunk):
        wid = _wid()
        lane = lax.iota(jnp.int32, 16)

        def one_direction(ids_hbm, out_hbm):
            pltpu.sync_copy(ids_hbm.at[pl.ds(wid * EPW, EPW)], chunk)

            def init_body(i, _):
                tbl[pl.ds(i * 16, 16)] = jnp.full((16,), jnp.int32(-1),
                                                  jnp.int32)
                return 0

            lax.fori_loop(0, NPAD // 16, init_body, 0)

            def scat_body(g, _):
                node = chunk[pl.ds(g * 16, 16)]
                ev = wid * EPW + g * 16 + lane
                for j in range(16):
                    plsc.store_scatter(tbl, [node], ev, mask=lane == j)
                return 0

            lax.fori_loop(0, grp, scat_body, 0)
            pltpu.sync_copy(tbl, out_hbm.at[wid])

        one_direction(src_hbm, ls_hbm)
        one_direction(dst_hbm, ld_hbm)

    return k(src, dst)


def _merge_body(ls_ref, ld_ref, lsc_ref, ldc_ref, lw4_ref, selw_ref, hd_ref):
    ms = jnp.max(ls_ref[...], axis=0)
    md = jnp.max(ld_ref[...], axis=0)
    lsc = jnp.maximum(ms, 0)
    ldc = jnp.maximum(md, 0)
    lwin = jnp.where(md >= 0, ldc, lsc)
    lsc_ref[...] = lsc
    ldc_ref[...] = ldc
    lw4_ref[...] = lax.shift_right_logical(lwin, 2)
    selw_ref[...] = lwin & 3
    hd_ref[...] = (md >= 0).astype(jnp.float32)


def _merge_tc(Ls, Ld):
    return pl.pallas_call(
        _merge_body,
        out_shape=[
            jax.ShapeDtypeStruct((NPAD,), jnp.int32),
            jax.ShapeDtypeStruct((NPAD,), jnp.int32),
            jax.ShapeDtypeStruct((NPAD,), jnp.int32),
            jax.ShapeDtypeStruct((NPAD,), jnp.int32),
            jax.ShapeDtypeStruct((NPAD,), jnp.float32),
        ],
    )(Ls, Ld)


def _sc_node_gathers(x_src, x_dst, Z, lsc, ldc, lw4):
    C = 136
    nchunk = RPW // C

    @functools.partial(
        pl.kernel,
        mesh=_mesh(),
        compiler_params=pltpu.CompilerParams(needs_layout_passes=False),
        out_type=[jax.ShapeDtypeStruct((NPAD, D), jnp.float32)] * 3,
        scratch_types=[
            pltpu.VMEM((RPW,), jnp.int32),
            pltpu.VMEM((RPW,), jnp.int32),
            pltpu.VMEM((RPW,), jnp.int32),
        ]
        + [pltpu.VMEM((C, D), jnp.float32)] * 6
        + [pltpu.SemaphoreType.DMA] * 6,
    )
    def k(xs_hbm, xd_hbm, z_hbm, lsc_hbm, ldc_hbm, lw4_hbm,
          wxs_hbm, wxd_hbm, wzw_hbm, idx_s, idx_d, idx_w,
          b0, b1, b2, b3, b4, b5, s0, s1, s2, s3, s4, s5):
        wid = _wid()
        base = wid * RPW
        pltpu.sync_copy(lsc_hbm.at[pl.ds(base, RPW)], idx_s)
        pltpu.sync_copy(ldc_hbm.at[pl.ds(base, RPW)], idx_d)
        pltpu.sync_copy(lw4_hbm.at[pl.ds(base, RPW)], idx_w)
        _ring_gather_multi(
            [(xs_hbm, idx_s, wxs_hbm, b0, b1, s0, s1),
             (xd_hbm, idx_d, wxd_hbm, b2, b3, s2, s3),
             (z_hbm, idx_w, wzw_hbm, b4, b5, s4, s5)],
            base, C, nchunk)

    return k(x_src, x_dst, Z, lsc, ldc, lw4)


E2 = E // 2
EPW2 = E2 // NTILES


def _sc_edge_gathers(A, B, src_h, dst_h):
    C = 200
    nchunk = EPW2 // C

    @functools.partial(
        pl.kernel,
        mesh=_mesh(),
        compiler_params=pltpu.CompilerParams(needs_layout_passes=False),
        out_type=[jax.ShapeDtypeStruct((E2, D), jnp.int32)] * 2,
        scratch_types=[
            pltpu.VMEM((EPW2,), jnp.int32),
            pltpu.VMEM((EPW2,), jnp.int32),
        ]
        + [pltpu.VMEM((C, D), jnp.int32)] * 4
        + [pltpu.SemaphoreType.DMA] * 4,
    )
    def k(a_hbm, b_hbm, src_hbm, dst_hbm, gs_hbm, gd_hbm, idx_s, idx_d,
          b0, b1, b2, b3, s0, s1, s2, s3):
        wid = _wid()
        base = wid * EPW2
        pltpu.sync_copy(src_hbm.at[pl.ds(base, EPW2)], idx_s)
        pltpu.sync_copy(dst_hbm.at[pl.ds(base, EPW2)], idx_d)
        _ring_gather_multi(
            [(a_hbm, idx_s, gs_hbm, b0, b1, s0, s1),
             (b_hbm, idx_d, gd_hbm, b2, b3, s2, s3)],
            base, C, nchunk)

    return k(A, B, src_h, dst_h)


def _sel32(z, sel):
    out = jnp.zeros((z.shape[0], 2 * D_EDGE), jnp.float32)
    for kk in range(4):
        out += jnp.where(sel == kk, z[:, 32 * kk:32 * kk + 32], 0.0)
    return out


def _node_enc_body(wxs_ref, wxd_ref, wzw_ref, selw_ref,
                   hd_ref, wenc_ref, wcat_ref, a_ref, b_ref):
    S = jax.nn.relu(
        jnp.dot(wxs_ref[...], wenc_ref[...], preferred_element_type=jnp.float32))
    T = jax.nn.relu(
        jnp.dot(wxd_ref[...], wenc_ref[...], preferred_element_type=jnp.float32))
    zw = _sel32(wzw_ref[...], selw_ref[...])
    WEH = jnp.dot(zw, wcat_ref[...], preferred_element_type=jnp.float32)
    hd = hd_ref[...]
    WH = jnp.where(hd > 0.5, T, S) + WEH

    def rnd(x):
        return lax.bitcast_convert_type(x, jnp.int32) + 0x8000

    wh_hi = rnd(WH) & jnp.int32(-65536)
    a_ref[...] = wh_hi | lax.shift_right_logical(rnd(S), 16)
    b_ref[...] = wh_hi | lax.shift_right_logical(rnd(T), 16)


def _node_enc(wx_src, wx_dst, wzw, selw, has_dst, W_enc, Wcat, n_rows):
    blk = 3128
    grid = (n_rows // blk,)
    return pl.pallas_call(
        _node_enc_body,
        grid=grid,
        in_specs=[
            pl.BlockSpec((blk, D), lambda i: (i, 0)),
            pl.BlockSpec((blk, D), lambda i: (i, 0)),
            pl.BlockSpec((blk, D), lambda i: (i, 0)),
            pl.BlockSpec((blk, 1), lambda i: (i, 0)),
            pl.BlockSpec((blk, 1), lambda i: (i, 0)),
            pl.BlockSpec((D, D), lambda i: (0, 0)),
            pl.BlockSpec((2 * D_EDGE, D), lambda i: (0, 0)),
        ],
        out_specs=[
            pl.BlockSpec((blk, D), lambda i: (i, 0)),
            pl.BlockSpec((blk, D), lambda i: (i, 0)),
        ],
        out_shape=[
            jax.ShapeDtypeStruct((n_rows, D), jnp.int32),
            jax.ShapeDtypeStruct((n_rows, D), jnp.int32),
        ],
    )(wx_src, wx_dst, wzw, selw, has_dst, W_enc, Wcat)


def _final_body(gs_ref, gd_ref, msg_ref, ef_ref, et_ref, wm_ref, we_ref,
                wdec_ref, tb_ref, out_ref, acc_ref, accv_ref):
    i = pl.program_id(0)

    @pl.when(i == 0)
    def _():
        acc_ref[0] = 0.0
        accv_ref[...] = jnp.zeros_like(accv_ref)

    EH = (jnp.dot(msg_ref[...], wm_ref[...], preferred_element_type=jnp.float32)
          + jnp.dot(ef_ref[...], we_ref[...], preferred_element_type=jnp.float32))
    gs = gs_ref[...]
    gd = gd_ref[...]

    def lo(w):
        return lax.bitcast_convert_type(lax.shift_left(w, 16), jnp.float32)

    def hi(w):
        return lax.bitcast_convert_type(w & jnp.int32(-65536), jnp.float32)

    h_src = lo(gs) + EH
    h_dst = lo(gd) + EH
    hdw = jnp.dot(h_dst, wdec_ref[...], preferred_element_type=jnp.float32)
    ones_d = jnp.ones((D,), jnp.float32)
    score = jnp.dot(h_src * hdw, ones_d, preferred_element_type=jnp.float32)
    et = et_ref[0, 0, :]
    bias = jnp.zeros_like(score)
    for k in range(N_TYPES):
        bias += jnp.where(et == k, tb_ref[k], 0.0)
    score = score + bias
    sp = jnp.maximum(-score, 0.0) + jnp.log1p(jnp.exp(-jnp.abs(score)))
    ds = h_src - hi(gs)
    dd = h_dst - hi(gd)
    q = ds * ds + dd * dd
    blkn = q.shape[0]
    accv_ref[...] += jnp.dot(jnp.ones((1, blkn), jnp.float32), q,
                             preferred_element_type=jnp.float32)
    acc_ref[0] += jnp.sum(sp)

    @pl.when(i == pl.num_programs(0) - 1)
    def _():
        out_ref[0] = acc_ref[0]
        out_ref[1] = jnp.sum(accv_ref[...])


def _final(Gs, Gd, msg, ef, edge_type, W_msg, W_ef, W_dec, type_bias):
    blk = 1600
    grid = (E2 // blk,)
    et3 = edge_type.astype(jnp.int32).reshape(E2 // blk, 1, blk)
    return pl.pallas_call(
        _final_body,
        grid=grid,
        in_specs=[
            pl.BlockSpec((blk, D), lambda i: (i, 0)),
            pl.BlockSpec((blk, D), lambda i: (i, 0)),
            pl.BlockSpec((blk, D_EDGE), lambda i: (i, 0)),
            pl.BlockSpec((blk, D_EDGE), lambda i: (i, 0)),
            pl.BlockSpec((1, 1, blk), lambda i: (i, 0, 0)),
            pl.BlockSpec((D_EDGE, D), lambda i: (0, 0)),
            pl.BlockSpec((D_EDGE, D), lambda i: (0, 0)),
            pl.BlockSpec((D, D), lambda i: (0, 0)),
            pl.BlockSpec(memory_space=pltpu.SMEM),
        ],
        out_specs=pl.BlockSpec(memory_space=pltpu.SMEM),
        out_shape=jax.ShapeDtypeStruct((2,), jnp.float32),
        scratch_shapes=[pltpu.SMEM((2,), jnp.float32),
                        pltpu.VMEM((1, D), jnp.float32)],
        compiler_params=pltpu.CompilerParams(
            dimension_semantics=("arbitrary",)),
    )(Gs, Gd, msg, ef, et3, W_msg, W_ef, W_dec, type_bias)


def kernel(x_src, x_dst, msg, edge_feats, W_enc, W_msg, W_ef, W_dec, type_bias,
           last_h_storage, src, dst, t, edge_type):
    src = src.astype(jnp.int32)
    dst = dst.astype(jnp.int32)

    Ls, Ld = _sc_winner_tables(src, dst)
    lsc, ldc, lw4, selw, hd = _merge_tc(Ls, Ld)

    Z = jnp.concatenate([msg, edge_feats], axis=1).reshape(E // 4, 4 * 2 * D_EDGE)
    Wcat = jnp.concatenate([W_msg, W_ef], axis=0)

    wx_src, wx_dst, wzw = _sc_node_gathers(x_src, x_dst, Z, lsc, ldc, lw4)

    A, B = _node_enc(wx_src, wx_dst, wzw, selw.reshape(NPAD, 1),
                     hd.reshape(NPAD, 1), W_enc, Wcat, NPAD)

    acc = None
    for h in range(2):
        sl = slice(h * E2, (h + 1) * E2)
        Gs, Gd = _sc_edge_gathers(A, B, src[sl], dst[sl])
        part = _final(Gs, Gd, msg[sl], edge_feats[sl], edge_type[sl],
                      W_msg, W_ef, W_dec, type_bias)
        acc = part if acc is None else acc + part
    return (acc[0] / E + 0.1 * (acc[1] / (E * D))).reshape(1)


# speedup vs baseline: 1.0000x; 1.0000x over previous
import functools

import jax
import jax.numpy as jnp
from jax import lax
from jax.experimental import pallas as pl
from jax.experimental.pallas import tpu as pltpu
from jax.experimental.pallas import tpu_sc as plsc

NUM_NODES = 100000
E = 320000
D = 128
D_EDGE = 16
N_TYPES = 8

NTILES = 32
NPAD = 100096
EPW = E // NTILES
RPW = NPAD // NTILES


def _mesh():
    return plsc.VectorSubcoreMesh(core_axis_name="c", subcore_axis_name="s")


def _wid():
    return lax.axis_index("s") * 2 + lax.axis_index("c")


def _ring_gather_multi(jobs, base, C, nchunk):
    def sg(tab, idxref, j, buf, sem):
        pltpu.async_copy(tab.at[idxref.at[pl.ds(j * C, C)]], buf, sem)

    def wg(tab, buf, sem):
        pltpu.make_async_copy(tab.at[pl.ds(0, C)], buf, sem).wait()

    def out_cp(out, j, buf):
        pltpu.sync_copy(buf, out.at[pl.ds(base + j * C, C)])

    for tab, idx, out, ba, bb, sa, sb in jobs:
        sg(tab, idx, 0, ba, sa)
        sg(tab, idx, 1, bb, sb)
    npair = (nchunk - 2) // 2 if nchunk % 2 == 0 else (nchunk - 3) // 2

    def body(j2, _):
        j = 2 * j2
        for tab, idx, out, ba, bb, sa, sb in jobs:
            wg(tab, ba, sa)
            out_cp(out, j, ba)
            sg(tab, idx, j + 2, ba, sa)
            wg(tab, bb, sb)
            out_cp(out, j + 1, bb)
            sg(tab, idx, j + 3, bb, sb)
        return 0

    lax.fori_loop(0, npair, body, 0)
    for tab, idx, out, ba, bb, sa, sb in jobs:
        if nchunk % 2 == 0:
            wg(tab, ba, sa)
            out_cp(out, nchunk - 2, ba)
            wg(tab, bb, sb)
            out_cp(out, nchunk - 1, bb)
        else:
            wg(tab, ba, sa)
            out_cp(out, nchunk - 3, ba)
            sg(tab, idx, nchunk - 1, ba, sa)
            wg(tab, bb, sb)
            out_cp(out, nchunk - 2, bb)
            wg(tab, ba, sa)
            out_cp(out, nchunk - 1, ba)


def _sc_winner_tables(src, dst):
    grp = EPW // 16

    @functools.partial(
        pl.kernel,
        mesh=_mesh(),
        compiler_params=pltpu.CompilerParams(needs_layout_passes=False),
        out_type=[
            jax.ShapeDtypeStruct((NTILES, NPAD), jnp.int32),
            jax.ShapeDtypeStruct((NTILES, NPAD), jnp.int32),
        ],
        scratch_types=[
            pltpu.VMEM((NPAD,), jnp.int32),
            pltpu.VMEM((EPW,), jnp.int32),
        ],
    )
    def k(src_hbm, dst_hbm, ls_hbm, ld_hbm, tbl, chunk):
        wid = _wid()
        lane = lax.iota(jnp.int32, 16)

        def one_direction(ids_hbm, out_hbm):
            pltpu.sync_copy(ids_hbm.at[pl.ds(wid * EPW, EPW)], chunk)

            def init_body(i, _):
                tbl[pl.ds(i * 16, 16)] = jnp.full((16,), jnp.int32(-1),
                                                  jnp.int32)
                return 0

            lax.fori_loop(0, NPAD // 16, init_body, 0)

            def scat_body(g, _):
                node = chunk[pl.ds(g * 16, 16)]
                ev = wid * EPW + g * 16 + lane
                for j in range(16):
                    plsc.store_scatter(tbl, [node], ev, mask=lane == j)
                return 0

            lax.fori_loop(0, grp, scat_body, 0)
            pltpu.sync_copy(tbl, out_hbm.at[wid])

        one_direction(src_hbm, ls_hbm)
        one_direction(dst_hbm, ld_hbm)

    return k(src, dst)


def _merge_body(ls_ref, ld_ref, lsc_ref, ldc_ref, lw4_ref, selw_ref, hd_ref):
    ms = jnp.max(ls_ref[...], axis=0)
    md = jnp.max(ld_ref[...], axis=0)
    lsc = jnp.maximum(ms, 0)
    ldc = jnp.maximum(md, 0)
    lwin = jnp.where(md >= 0, ldc, lsc)
    lsc_ref[...] = lsc
    ldc_ref[...] = ldc
    lw4_ref[...] = lax.shift_right_logical(lwin, 2)
    selw_ref[...] = lwin & 3
    hd_ref[...] = (md >= 0).astype(jnp.float32)


def _merge_tc(Ls, Ld):
    return pl.pallas_call(
        _merge_body,
        out_shape=[
            jax.ShapeDtypeStruct((NPAD,), jnp.int32),
            jax.ShapeDtypeStruct((NPAD,), jnp.int32),
            jax.ShapeDtypeStruct((NPAD,), jnp.int32),
            jax.ShapeDtypeStruct((NPAD,), jnp.int32),
            jax.ShapeDtypeStruct((NPAD,), jnp.float32),
        ],
    )(Ls, Ld)


def _sc_node_gathers(x_src, x_dst, Z, lsc, ldc, lw4):
    C = 136
    nchunk = RPW // C

    @functools.partial(
        pl.kernel,
        mesh=_mesh(),
        compiler_params=pltpu.CompilerParams(needs_layout_passes=False),
        out_type=[jax.ShapeDtypeStruct((NPAD, D), jnp.float32)] * 3,
        scratch_types=[
            pltpu.VMEM((RPW,), jnp.int32),
            pltpu.VMEM((RPW,), jnp.int32),
            pltpu.VMEM((RPW,), jnp.int32),
        ]
        + [pltpu.VMEM((C, D), jnp.float32)] * 6
        + [pltpu.SemaphoreType.DMA] * 6,
    )
    def k(xs_hbm, xd_hbm, z_hbm, lsc_hbm, ldc_hbm, lw4_hbm,
          wxs_hbm, wxd_hbm, wzw_hbm, idx_s, idx_d, idx_w,
          b0, b1, b2, b3, b4, b5, s0, s1, s2, s3, s4, s5):
        wid = _wid()
        base = wid * RPW
        pltpu.sync_copy(lsc_hbm.at[pl.ds(base, RPW)], idx_s)
        pltpu.sync_copy(ldc_hbm.at[pl.ds(base, RPW)], idx_d)
        pltpu.sync_copy(lw4_hbm.at[pl.ds(base, RPW)], idx_w)
        _ring_gather_multi(
            [(xs_hbm, idx_s, wxs_hbm, b0, b1, s0, s1),
             (xd_hbm, idx_d, wxd_hbm, b2, b3, s2, s3),
             (z_hbm, idx_w, wzw_hbm, b4, b5, s4, s5)],
            base, C, nchunk)

    return k(x_src, x_dst, Z, lsc, ldc, lw4)


E2 = E // 2
EPW2 = E2 // NTILES


def _sc_edge_gathers(A, B, src_h, dst_h):
    C = 200
    nchunk = EPW2 // C

    @functools.partial(
        pl.kernel,
        mesh=_mesh(),
        compiler_params=pltpu.CompilerParams(needs_layout_passes=False),
        out_type=[jax.ShapeDtypeStruct((E2, D), jnp.int32)] * 2,
        scratch_types=[
            pltpu.VMEM((EPW2,), jnp.int32),
            pltpu.VMEM((EPW2,), jnp.int32),
        ]
        + [pltpu.VMEM((C, D), jnp.int32)] * 4
        + [pltpu.SemaphoreType.DMA] * 4,
    )
    def k(a_hbm, b_hbm, src_hbm, dst_hbm, gs_hbm, gd_hbm, idx_s, idx_d,
          b0, b1, b2, b3, s0, s1, s2, s3):
        wid = _wid()
        base = wid * EPW2
        pltpu.sync_copy(src_hbm.at[pl.ds(base, EPW2)], idx_s)
        pltpu.sync_copy(dst_hbm.at[pl.ds(base, EPW2)], idx_d)
        _ring_gather_multi(
            [(a_hbm, idx_s, gs_hbm, b0, b1, s0, s1),
             (b_hbm, idx_d, gd_hbm, b2, b3, s2, s3)],
            base, C, nchunk)

    return k(A, B, src_h, dst_h)


def _sel32(z, sel):
    out = jnp.zeros((z.shape[0], 2 * D_EDGE), jnp.float32)
    for kk in range(4):
        out += jnp.where(sel == kk, z[:, 32 * kk:32 * kk + 32], 0.0)
    return out


def _node_enc_body(wxs_ref, wxd_ref, wzw_ref, selw_ref,
                   hd_ref, wenc_ref, wcat_ref, a_ref, b_ref):
    S = jax.nn.relu(
        jnp.dot(wxs_ref[...], wenc_ref[...], preferred_element_type=jnp.float32))
    T = jax.nn.relu(
        jnp.dot(wxd_ref[...], wenc_ref[...], preferred_element_type=jnp.float32))
    zw = _sel32(wzw_ref[...], selw_ref[...])
    WEH = jnp.dot(zw, wcat_ref[...], preferred_element_type=jnp.float32)
    hd = hd_ref[...]
    WH = jnp.where(hd > 0.5, T, S) + WEH

    def rnd(x):
        return lax.bitcast_convert_type(x, jnp.int32) + 0x8000

    wh_hi = rnd(WH) & jnp.int32(-65536)
    a_ref[...] = wh_hi | lax.shift_right_logical(rnd(S), 16)
    b_ref[...] = wh_hi | lax.shift_right_logical(rnd(T), 16)


def _node_enc(wx_src, wx_dst, wzw, selw, has_dst, W_enc, Wcat, n_rows):
    blk = 3128
    grid = (n_rows // blk,)
    return pl.pallas_call(
        _node_enc_body,
        grid=grid,
        in_specs=[
            pl.BlockSpec((blk, D), lambda i: (i, 0)),
            pl.BlockSpec((blk, D), lambda i: (i, 0)),
            pl.BlockSpec((blk, D), lambda i: (i, 0)),
            pl.BlockSpec((blk, 1), lambda i: (i, 0)),
            pl.BlockSpec((blk, 1), lambda i: (i, 0)),
            pl.BlockSpec((D, D), lambda i: (0, 0)),
            pl.BlockSpec((2 * D_EDGE, D), lambda i: (0, 0)),
        ],
        out_specs=[
            pl.BlockSpec((blk, D), lambda i: (i, 0)),
            pl.BlockSpec((blk, D), lambda i: (i, 0)),
        ],
        out_shape=[
            jax.ShapeDtypeStruct((n_rows, D), jnp.int32),
            jax.ShapeDtypeStruct((n_rows, D), jnp.int32),
        ],
    )(wx_src, wx_dst, wzw, selw, has_dst, W_enc, Wcat)


def _final_body(gs_ref, gd_ref, msg_ref, ef_ref, et_ref, wm_ref, we_ref,
                wdec_ref, tb_ref, out_ref, acc_ref, accv_ref):
    i = pl.program_id(0)

    @pl.when(i == 0)
    def _():
        acc_ref[0] = 0.0
        accv_ref[...] = jnp.zeros_like(accv_ref)

    EH = (jnp.dot(msg_ref[...], wm_ref[...], preferred_element_type=jnp.float32)
          + jnp.dot(ef_ref[...], we_ref[...], preferred_element_type=jnp.float32))
    gs = gs_ref[...]
    gd = gd_ref[...]

    def lo(w):
        return lax.bitcast_convert_type(lax.shift_left(w, 16), jnp.float32)

    def hi(w):
        return lax.bitcast_convert_type(w & jnp.int32(-65536), jnp.float32)

    h_src = lo(gs) + EH
    h_dst = lo(gd) + EH
    hdw = jnp.dot(h_dst, wdec_ref[...], preferred_element_type=jnp.float32)
    ones_d = jnp.ones((D,), jnp.float32)
    score = jnp.dot(h_src * hdw, ones_d, preferred_element_type=jnp.float32)
    et = et_ref[0, 0, :]
    bias = jnp.zeros_like(score)
    for k in range(N_TYPES):
        bias += jnp.where(et == k, tb_ref[k], 0.0)
    score = score + bias
    sp = jnp.maximum(-score, 0.0) + jnp.log1p(jnp.exp(-jnp.abs(score)))
    ds = h_src - hi(gs)
    dd = h_dst - hi(gd)
    q = ds * ds + dd * dd
    blkn = q.shape[0]
    accv_ref[...] += jnp.dot(jnp.ones((1, blkn), jnp.float32), q,
                             preferred_element_type=jnp.float32)
    acc_ref[0] += jnp.sum(sp)

    @pl.when(i == pl.num_programs(0) - 1)
    def _():
        out_ref[0] = acc_ref[0]
        out_ref[1] = jnp.sum(accv_ref[...])


def _final(Gs, Gd, msg, ef, edge_type, W_msg, W_ef, W_dec, type_bias):
    blk = 1600
    grid = (E2 // blk,)
    et3 = edge_type.astype(jnp.int32).reshape(E2 // blk, 1, blk)
    return pl.pallas_call(
        _final_body,
        grid=grid,
        in_specs=[
            pl.BlockSpec((blk, D), lambda i: (i, 0)),
            pl.BlockSpec((blk, D), lambda i: (i, 0)),
            pl.BlockSpec((blk, D_EDGE), lambda i: (i, 0)),
            pl.BlockSpec((blk, D_EDGE), lambda i: (i, 0)),
            pl.BlockSpec((1, 1, blk), lambda i: (i, 0, 0)),
            pl.BlockSpec((D_EDGE, D), lambda i: (0, 0)),
            pl.BlockSpec((D_EDGE, D), lambda i: (0, 0)),
            pl.BlockSpec((D, D), lambda i: (0, 0)),
            pl.BlockSpec(memory_space=pltpu.SMEM),
        ],
        out_specs=pl.BlockSpec(memory_space=pltpu.SMEM),
        out_shape=jax.ShapeDtypeStruct((2,), jnp.float32),
        scratch_shapes=[pltpu.SMEM((2,), jnp.float32),
                        pltpu.VMEM((1, D), jnp.float32)],
        compiler_params=pltpu.CompilerParams(
            dimension_semantics=("arbitrary",)),
    )(Gs, Gd, msg, ef, et3, W_msg, W_ef, W_dec, type_bias)


def kernel(x_src, x_dst, msg, edge_feats, W_enc, W_msg, W_ef, W_dec, type_bias,
           last_h_storage, src, dst, t, edge_type):
    src = src.astype(jnp.int32)
    dst = dst.astype(jnp.int32)

    Ls, Ld = _sc_winner_tables(src, dst)
    lsc, ldc, lw4, selw, hd = _merge_tc(Ls, Ld)

    Z = jnp.concatenate([msg, edge_feats], axis=1).reshape(E // 4, 4 * 2 * D_EDGE)
    Wcat = jnp.concatenate([W_msg, W_ef], axis=0)

    wx_src, wx_dst, wzw = _sc_node_gathers(x_src, x_dst, Z, lsc, ldc, lw4)

    A, B = _node_enc(wx_src, wx_dst, wzw, selw.reshape(NPAD, 1),
                     hd.reshape(NPAD, 1), W_enc, Wcat, NPAD)

    acc = None
    for h in range(2):
        sl = slice(h * E2, (h + 1) * E2)
        Gs, Gd = _sc_edge_gathers(A, B, src[sl], dst[sl])
        part = _final(Gs, Gd, msg[sl], edge_feats[sl], edge_type[sl],
                      W_msg, W_ef, W_dec, type_bias)
        acc = part if acc is None else acc + part
    return (acc[0] / E + 0.1 * (acc[1] / (E * D))).reshape(1)


# speedup vs baseline: 1.0152x; 1.0152x over previous
import functools

import jax
import jax.numpy as jnp
from jax import lax
from jax.experimental import pallas as pl
from jax.experimental.pallas import tpu as pltpu
from jax.experimental.pallas import tpu_sc as plsc

NUM_NODES = 100000
E = 320000
D = 128
D_EDGE = 16
N_TYPES = 8

NTILES = 32
NPAD = 100096
EPW = E // NTILES
RPW = NPAD // NTILES


def _mesh():
    return plsc.VectorSubcoreMesh(core_axis_name="c", subcore_axis_name="s")


def _wid():
    return lax.axis_index("s") * 2 + lax.axis_index("c")


def _ring_gather_multi(jobs, base, C, nchunk):
    def sg(tab, idxref, j, buf, sem):
        pltpu.async_copy(tab.at[idxref.at[pl.ds(j * C, C)]], buf, sem)

    def wg(tab, buf, sem):
        pltpu.make_async_copy(tab.at[pl.ds(0, C)], buf, sem).wait()

    def out_cp(out, j, buf):
        pltpu.sync_copy(buf, out.at[pl.ds(base + j * C, C)])

    for tab, idx, out, ba, bb, sa, sb in jobs:
        sg(tab, idx, 0, ba, sa)
        sg(tab, idx, 1, bb, sb)
    npair = (nchunk - 2) // 2 if nchunk % 2 == 0 else (nchunk - 3) // 2

    def body(j2, _):
        j = 2 * j2
        for tab, idx, out, ba, bb, sa, sb in jobs:
            wg(tab, ba, sa)
            out_cp(out, j, ba)
            sg(tab, idx, j + 2, ba, sa)
            wg(tab, bb, sb)
            out_cp(out, j + 1, bb)
            sg(tab, idx, j + 3, bb, sb)
        return 0

    lax.fori_loop(0, npair, body, 0)
    for tab, idx, out, ba, bb, sa, sb in jobs:
        if nchunk % 2 == 0:
            wg(tab, ba, sa)
            out_cp(out, nchunk - 2, ba)
            wg(tab, bb, sb)
            out_cp(out, nchunk - 1, bb)
        else:
            wg(tab, ba, sa)
            out_cp(out, nchunk - 3, ba)
            sg(tab, idx, nchunk - 1, ba, sa)
            wg(tab, bb, sb)
            out_cp(out, nchunk - 2, bb)
            wg(tab, ba, sa)
            out_cp(out, nchunk - 1, ba)


def _sc_winner_tables(src, dst):
    grp = EPW // 16

    @functools.partial(
        pl.kernel,
        mesh=_mesh(),
        compiler_params=pltpu.CompilerParams(needs_layout_passes=False),
        out_type=[
            jax.ShapeDtypeStruct((NTILES, NPAD), jnp.int32),
            jax.ShapeDtypeStruct((NTILES, NPAD), jnp.int32),
        ],
        scratch_types=[
            pltpu.VMEM((NPAD,), jnp.int32),
            pltpu.VMEM((EPW,), jnp.int32),
        ],
    )
    def k(src_hbm, dst_hbm, ls_hbm, ld_hbm, tbl, chunk):
        wid = _wid()
        lane = lax.iota(jnp.int32, 16)

        def one_direction(ids_hbm, out_hbm):
            pltpu.sync_copy(ids_hbm.at[pl.ds(wid * EPW, EPW)], chunk)

            def init_body(i, _):
                tbl[pl.ds(i * 16, 16)] = jnp.full((16,), jnp.int32(-1),
                                                  jnp.int32)
                return 0

            lax.fori_loop(0, NPAD // 16, init_body, 0)

            def scat_body(g, _):
                node = chunk[pl.ds(g * 16, 16)]
                ev = wid * EPW + g * 16 + lane
                for j in range(16):
                    plsc.store_scatter(tbl, [node], ev, mask=lane == j)
                return 0

            lax.fori_loop(0, grp, scat_body, 0)
            pltpu.sync_copy(tbl, out_hbm.at[wid])

        one_direction(src_hbm, ls_hbm)
        one_direction(dst_hbm, ld_hbm)

    return k(src, dst)


def _merge_body(ls_ref, ld_ref, lsc_ref, ldc_ref, lw4_ref, selw_ref, hd_ref):
    ms = jnp.max(ls_ref[...], axis=0)
    md = jnp.max(ld_ref[...], axis=0)
    lsc = jnp.maximum(ms, 0)
    ldc = jnp.maximum(md, 0)
    lwin = jnp.where(md >= 0, ldc, lsc)
    lsc_ref[...] = lsc
    ldc_ref[...] = ldc
    lw4_ref[...] = lax.shift_right_logical(lwin, 2)
    selw_ref[...] = lwin & 3
    hd_ref[...] = (md >= 0).astype(jnp.float32)


def _merge_tc(Ls, Ld):
    return pl.pallas_call(
        _merge_body,
        out_shape=[
            jax.ShapeDtypeStruct((NPAD,), jnp.int32),
            jax.ShapeDtypeStruct((NPAD,), jnp.int32),
            jax.ShapeDtypeStruct((NPAD,), jnp.int32),
            jax.ShapeDtypeStruct((NPAD,), jnp.int32),
            jax.ShapeDtypeStruct((NPAD,), jnp.float32),
        ],
    )(Ls, Ld)


def _sc_node_gathers(x_src, x_dst, Z, lsc, ldc, lw4):
    C = 136
    nchunk = RPW // C

    @functools.partial(
        pl.kernel,
        mesh=_mesh(),
        compiler_params=pltpu.CompilerParams(needs_layout_passes=False),
        out_type=[jax.ShapeDtypeStruct((NPAD, D), jnp.float32)] * 3,
        scratch_types=[
            pltpu.VMEM((RPW,), jnp.int32),
            pltpu.VMEM((RPW,), jnp.int32),
            pltpu.VMEM((RPW,), jnp.int32),
        ]
        + [pltpu.VMEM((C, D), jnp.float32)] * 6
        + [pltpu.SemaphoreType.DMA] * 6,
    )
    def k(xs_hbm, xd_hbm, z_hbm, lsc_hbm, ldc_hbm, lw4_hbm,
          wxs_hbm, wxd_hbm, wzw_hbm, idx_s, idx_d, idx_w,
          b0, b1, b2, b3, b4, b5, s0, s1, s2, s3, s4, s5):
        wid = _wid()
        base = wid * RPW
        pltpu.sync_copy(lsc_hbm.at[pl.ds(base, RPW)], idx_s)
        pltpu.sync_copy(ldc_hbm.at[pl.ds(base, RPW)], idx_d)
        pltpu.sync_copy(lw4_hbm.at[pl.ds(base, RPW)], idx_w)
        _ring_gather_multi(
            [(xs_hbm, idx_s, wxs_hbm, b0, b1, s0, s1),
             (xd_hbm, idx_d, wxd_hbm, b2, b3, s2, s3),
             (z_hbm, idx_w, wzw_hbm, b4, b5, s4, s5)],
            base, C, nchunk)

    return k(x_src, x_dst, Z, lsc, ldc, lw4)


E2 = E
EPW2 = E2 // NTILES


def _sc_edge_gathers(A, B, src_h, dst_h):
    C = 200
    nchunk = EPW2 // C

    @functools.partial(
        pl.kernel,
        mesh=_mesh(),
        compiler_params=pltpu.CompilerParams(needs_layout_passes=False),
        out_type=[jax.ShapeDtypeStruct((E2, D), jnp.int32)] * 2,
        scratch_types=[
            pltpu.VMEM((EPW2,), jnp.int32),
            pltpu.VMEM((EPW2,), jnp.int32),
        ]
        + [pltpu.VMEM((C, D), jnp.int32)] * 4
        + [pltpu.SemaphoreType.DMA] * 4,
    )
    def k(a_hbm, b_hbm, src_hbm, dst_hbm, gs_hbm, gd_hbm, idx_s, idx_d,
          b0, b1, b2, b3, s0, s1, s2, s3):
        wid = _wid()
        base = wid * EPW2
        pltpu.sync_copy(src_hbm.at[pl.ds(base, EPW2)], idx_s)
        pltpu.sync_copy(dst_hbm.at[pl.ds(base, EPW2)], idx_d)
        _ring_gather_multi(
            [(a_hbm, idx_s, gs_hbm, b0, b1, s0, s1),
             (b_hbm, idx_d, gd_hbm, b2, b3, s2, s3)],
            base, C, nchunk)

    return k(A, B, src_h, dst_h)


def _sel32(z, sel):
    out = jnp.zeros((z.shape[0], 2 * D_EDGE), jnp.float32)
    for kk in range(4):
        out += jnp.where(sel == kk, z[:, 32 * kk:32 * kk + 32], 0.0)
    return out


def _node_enc_body(wxs_ref, wxd_ref, wzw_ref, selw_ref,
                   hd_ref, wenc_ref, wcat_ref, a_ref, b_ref):
    S = jax.nn.relu(
        jnp.dot(wxs_ref[...], wenc_ref[...], preferred_element_type=jnp.float32))
    T = jax.nn.relu(
        jnp.dot(wxd_ref[...], wenc_ref[...], preferred_element_type=jnp.float32))
    zw = _sel32(wzw_ref[...], selw_ref[...])
    WEH = jnp.dot(zw, wcat_ref[...], preferred_element_type=jnp.float32)
    hd = hd_ref[...]
    WH = jnp.where(hd > 0.5, T, S) + WEH

    def rnd(x):
        return lax.bitcast_convert_type(x, jnp.int32) + 0x8000

    wh_hi = rnd(WH) & jnp.int32(-65536)
    a_ref[...] = wh_hi | lax.shift_right_logical(rnd(S), 16)
    b_ref[...] = wh_hi | lax.shift_right_logical(rnd(T), 16)


def _node_enc(wx_src, wx_dst, wzw, selw, has_dst, W_enc, Wcat, n_rows):
    blk = 3128
    grid = (n_rows // blk,)
    return pl.pallas_call(
        _node_enc_body,
        grid=grid,
        in_specs=[
            pl.BlockSpec((blk, D), lambda i: (i, 0)),
            pl.BlockSpec((blk, D), lambda i: (i, 0)),
            pl.BlockSpec((blk, D), lambda i: (i, 0)),
            pl.BlockSpec((blk, 1), lambda i: (i, 0)),
            pl.BlockSpec((blk, 1), lambda i: (i, 0)),
            pl.BlockSpec((D, D), lambda i: (0, 0)),
            pl.BlockSpec((2 * D_EDGE, D), lambda i: (0, 0)),
        ],
        out_specs=[
            pl.BlockSpec((blk, D), lambda i: (i, 0)),
            pl.BlockSpec((blk, D), lambda i: (i, 0)),
        ],
        out_shape=[
            jax.ShapeDtypeStruct((n_rows, D), jnp.int32),
            jax.ShapeDtypeStruct((n_rows, D), jnp.int32),
        ],
    )(wx_src, wx_dst, wzw, selw, has_dst, W_enc, Wcat)


def _final_body(gs_ref, gd_ref, msg_ref, ef_ref, et_ref, wm_ref, we_ref,
                wdec_ref, tb_ref, out_ref, acc_ref, accv_ref):
    i = pl.program_id(0)

    @pl.when(i == 0)
    def _():
        acc_ref[0] = 0.0
        accv_ref[...] = jnp.zeros_like(accv_ref)

    EH = (jnp.dot(msg_ref[...], wm_ref[...], preferred_element_type=jnp.float32)
          + jnp.dot(ef_ref[...], we_ref[...], preferred_element_type=jnp.float32))
    gs = gs_ref[...]
    gd = gd_ref[...]

    def lo(w):
        return lax.bitcast_convert_type(lax.shift_left(w, 16), jnp.float32)

    def hi(w):
        return lax.bitcast_convert_type(w & jnp.int32(-65536), jnp.float32)

    h_src = lo(gs) + EH
    h_dst = lo(gd) + EH
    hdw = jnp.dot(h_dst, wdec_ref[...], preferred_element_type=jnp.float32)
    ones_d = jnp.ones((D,), jnp.float32)
    score = jnp.dot(h_src * hdw, ones_d, preferred_element_type=jnp.float32)
    et = et_ref[0, 0, :]
    bias = jnp.zeros_like(score)
    for k in range(N_TYPES):
        bias += jnp.where(et == k, tb_ref[k], 0.0)
    score = score + bias
    sp = jnp.maximum(-score, 0.0) + jnp.log1p(jnp.exp(-jnp.abs(score)))
    ds = h_src - hi(gs)
    dd = h_dst - hi(gd)
    q = ds * ds + dd * dd
    blkn = q.shape[0]
    accv_ref[...] += jnp.dot(jnp.ones((1, blkn), jnp.float32), q,
                             preferred_element_type=jnp.float32)
    acc_ref[0] += jnp.sum(sp)

    @pl.when(i == pl.num_programs(0) - 1)
    def _():
        out_ref[0] = acc_ref[0]
        out_ref[1] = jnp.sum(accv_ref[...])


def _final(Gs, Gd, msg, ef, edge_type, W_msg, W_ef, W_dec, type_bias):
    blk = 1600
    grid = (E2 // blk,)
    et3 = edge_type.astype(jnp.int32).reshape(E2 // blk, 1, blk)
    return pl.pallas_call(
        _final_body,
        grid=grid,
        in_specs=[
            pl.BlockSpec((blk, D), lambda i: (i, 0)),
            pl.BlockSpec((blk, D), lambda i: (i, 0)),
            pl.BlockSpec((blk, D_EDGE), lambda i: (i, 0)),
            pl.BlockSpec((blk, D_EDGE), lambda i: (i, 0)),
            pl.BlockSpec((1, 1, blk), lambda i: (i, 0, 0)),
            pl.BlockSpec((D_EDGE, D), lambda i: (0, 0)),
            pl.BlockSpec((D_EDGE, D), lambda i: (0, 0)),
            pl.BlockSpec((D, D), lambda i: (0, 0)),
            pl.BlockSpec(memory_space=pltpu.SMEM),
        ],
        out_specs=pl.BlockSpec(memory_space=pltpu.SMEM),
        out_shape=jax.ShapeDtypeStruct((2,), jnp.float32),
        scratch_shapes=[pltpu.SMEM((2,), jnp.float32),
                        pltpu.VMEM((1, D), jnp.float32)],
        compiler_params=pltpu.CompilerParams(
            dimension_semantics=("arbitrary",)),
    )(Gs, Gd, msg, ef, et3, W_msg, W_ef, W_dec, type_bias)


def kernel(x_src, x_dst, msg, edge_feats, W_enc, W_msg, W_ef, W_dec, type_bias,
           last_h_storage, src, dst, t, edge_type):
    src = src.astype(jnp.int32)
    dst = dst.astype(jnp.int32)

    Ls, Ld = _sc_winner_tables(src, dst)
    lsc, ldc, lw4, selw, hd = _merge_tc(Ls, Ld)

    Z = jnp.concatenate([msg, edge_feats], axis=1).reshape(E // 4, 4 * 2 * D_EDGE)
    Wcat = jnp.concatenate([W_msg, W_ef], axis=0)

    wx_src, wx_dst, wzw = _sc_node_gathers(x_src, x_dst, Z, lsc, ldc, lw4)

    A, B = _node_enc(wx_src, wx_dst, wzw, selw.reshape(NPAD, 1),
                     hd.reshape(NPAD, 1), W_enc, Wcat, NPAD)

    Gs, Gd = _sc_edge_gathers(A, B, src, dst)
    acc = _final(Gs, Gd, msg, edge_feats, edge_type,
                 W_msg, W_ef, W_dec, type_bias)
    return (acc[0] / E + 0.1 * (acc[1] / (E * D))).reshape(1)
